# initial kernel scaffold (unmeasured)
import jax
import jax.numpy as jnp
from jax import lax
from jax.experimental import pallas as pl
from jax.experimental.pallas import tpu as pltpu


def kernel(
    x,
):
    def body(*refs):
        pass

    out_shape = jax.ShapeDtypeStruct(..., jnp.float32)
    return pl.pallas_call(body, out_shape=out_shape)(...)


# baseline (device time: 20687 ns/iter reference)
import jax
import jax.numpy as jnp
from jax import lax
from jax.experimental import pallas as pl
from jax.experimental.pallas import tpu as pltpu

N_DEV = 16


def kernel(x):
    m_rows, n_cols = x.shape

    def body(x_ref, out_ref, my_stats, gbuf, send_sems, recv_sems):
        me = lax.axis_index("i")

        xv = x_ref[...]
        m_loc = jnp.max(xv, axis=1, keepdims=True)
        s_loc = jnp.sum(jnp.exp(xv - m_loc), axis=1, keepdims=True)
        my_stats[...] = jnp.concatenate([m_loc, s_loc], axis=1).T

        for p in range(N_DEV):
            rdma = pltpu.make_async_remote_copy(
                src_ref=my_stats,
                dst_ref=gbuf.at[me],
                send_sem=send_sems.at[p],
                recv_sem=recv_sems.at[me],
                device_id=(p,),
                device_id_type=pl.DeviceIdType.MESH,
            )
            rdma.start()

        for p in range(N_DEV):
            recv = pltpu.make_async_remote_copy(
                src_ref=my_stats,
                dst_ref=gbuf.at[p],
                send_sem=send_sems.at[p],
                recv_sem=recv_sems.at[p],
                device_id=(p,),
                device_id_type=pl.DeviceIdType.MESH,
            )
            recv.wait_recv()

        g = gbuf[...]
        m_all = g[:, 0, :]
        s_all = g[:, 1, :]
        m_glob = jnp.max(m_all, axis=0)
        s_glob = jnp.sum(s_all * jnp.exp(m_all - m_glob[None, :]), axis=0)
        fin = jnp.stack([m_glob, 1.0 / s_glob]).T
        m_col = fin[:, 0:1]
        inv_s = fin[:, 1:2]

        out_ref[...] = jnp.exp(xv - m_col) * inv_s

        for p in range(N_DEV):
            drain = pltpu.make_async_remote_copy(
                src_ref=my_stats,
                dst_ref=gbuf.at[me],
                send_sem=send_sems.at[p],
                recv_sem=recv_sems.at[me],
                device_id=(p,),
                device_id_type=pl.DeviceIdType.MESH,
            )
            drain.wait_send()

    out_shape = jax.ShapeDtypeStruct((m_rows, n_cols), jnp.float32)
    return pl.pallas_call(
        body,
        out_shape=out_shape,
        in_specs=[pl.BlockSpec(memory_space=pltpu.VMEM)],
        out_specs=pl.BlockSpec(memory_space=pltpu.VMEM),
        scratch_shapes=[
            pltpu.VMEM((2, m_rows), jnp.float32),
            pltpu.VMEM((N_DEV, 2, m_rows), jnp.float32),
            pltpu.SemaphoreType.DMA((N_DEV,)),
            pltpu.SemaphoreType.DMA((N_DEV,)),
        ],
    )(x)


# device time: 13545 ns/iter; 1.5273x vs baseline; 1.5273x over previous
import jax
import jax.numpy as jnp
from jax import lax
from jax.experimental import pallas as pl
from jax.experimental.pallas import tpu as pltpu

N_DEV = 16


def kernel(x):
    m_rows, n_cols = x.shape

    def body(x_ref, out_ref, e_ref, my_stats, gbuf, send_sems, recv_sems):
        me = lax.axis_index("i")

        bsem = pltpu.get_barrier_semaphore()
        for p in range(N_DEV):
            pl.semaphore_signal(
                bsem, inc=1,
                device_id=(p,), device_id_type=pl.DeviceIdType.MESH,
            )

        xv = x_ref[...]
        m_loc = jnp.max(xv, axis=1, keepdims=True)
        e = jnp.exp(xv - m_loc)
        e_ref[...] = e
        s_loc = jnp.sum(e, axis=1, keepdims=True)
        my_stats[...] = jnp.concatenate([m_loc, s_loc], axis=1).T

        pl.semaphore_wait(bsem, N_DEV)

        for p in range(N_DEV):
            rdma = pltpu.make_async_remote_copy(
                src_ref=my_stats,
                dst_ref=gbuf.at[me],
                send_sem=send_sems.at[p],
                recv_sem=recv_sems.at[me],
                device_id=(p,),
                device_id_type=pl.DeviceIdType.MESH,
            )
            rdma.start()

        for p in range(N_DEV):
            recv = pltpu.make_async_remote_copy(
                src_ref=my_stats,
                dst_ref=gbuf.at[p],
                send_sem=send_sems.at[p],
                recv_sem=recv_sems.at[p],
                device_id=(p,),
                device_id_type=pl.DeviceIdType.MESH,
            )
            recv.wait_recv()

        g = gbuf[...]
        m_all = g[:, 0, :]
        s_all = g[:, 1, :]
        m_glob = jnp.max(m_all, axis=0)
        s_glob = jnp.sum(s_all * jnp.exp(m_all - m_glob[None, :]), axis=0)
        scale = jnp.exp(my_stats[0, :] - m_glob) / s_glob
        scale_col = scale[None, :].T

        out_ref[...] = e_ref[...] * scale_col

        for p in range(N_DEV):
            drain = pltpu.make_async_remote_copy(
                src_ref=my_stats,
                dst_ref=gbuf.at[me],
                send_sem=send_sems.at[p],
                recv_sem=recv_sems.at[me],
                device_id=(p,),
                device_id_type=pl.DeviceIdType.MESH,
            )
            drain.wait_send()

    out_shape = jax.ShapeDtypeStruct((m_rows, n_cols), jnp.float32)
    return pl.pallas_call(
        body,
        out_shape=out_shape,
        in_specs=[pl.BlockSpec(memory_space=pltpu.VMEM)],
        out_specs=pl.BlockSpec(memory_space=pltpu.VMEM),
        scratch_shapes=[
            pltpu.VMEM((m_rows, n_cols), jnp.float32),
            pltpu.VMEM((2, m_rows), jnp.float32),
            pltpu.VMEM((N_DEV, 2, m_rows), jnp.float32),
            pltpu.SemaphoreType.DMA((N_DEV,)),
            pltpu.SemaphoreType.DMA((N_DEV,)),
        ],
        compiler_params=pltpu.CompilerParams(collective_id=0),
    )(x)
